# baseline (speedup 1.0000x reference)
import functools

import jax
import jax.numpy as jnp
from jax import lax
from jax.experimental import pallas as pl
from jax.experimental.pallas import tpu as pltpu
from jax.experimental.pallas import tpu_sc as plsc

N = 50000
C = 128
K = 12
C_OUT = 128

NC = 2
NS = 16
NW = NC * NS
L = 16

CB = 2
NPW = 1568
STEPS = NPW // CB
NPAD = NW * NPW
G = CB * K
H = G // 2


NBUF = 4


def _sc_body(xt_hbm, idx0_hbm, idx1_hbm, m_hbm,
             idx0_v, idx1_v, rows0_v, rows1_v, out_v,
             sg0, sg1, sg2, sg3, so0, so1, so2, so3):
    wid = lax.axis_index("s") * NC + lax.axis_index("c")
    base = wid * NPW
    sem_g = (sg0, sg1, sg2, sg3)
    sem_out = (so0, so1, so2, so3)

    pltpu.sync_copy(idx0_hbm.at[pl.ds(base * K, (NPW + (NBUF - 1) * CB) * K)], idx0_v)
    pltpu.sync_copy(idx1_hbm.at[pl.ds(base * K, (NPW + (NBUF - 1) * CB) * K)], idx1_v)

    def issue_gathers(s, b):
        off = s * G
        pltpu.async_copy(xt_hbm.at[idx0_v.at[pl.ds(off, G)]], rows0_v.at[b], sem_g[b])
        pltpu.async_copy(xt_hbm.at[idx1_v.at[pl.ds(off, G)]], rows1_v.at[b], sem_g[b])

    def wait_gathers(b):
        pltpu.make_async_copy(xt_hbm.at[idx0_v.at[pl.ds(0, G)]], rows0_v.at[b], sem_g[b]).wait()
        pltpu.make_async_copy(xt_hbm.at[idx1_v.at[pl.ds(0, G)]], rows1_v.at[b], sem_g[b]).wait()

    for b in range(NBUF - 1):
        issue_gathers(b, b)

    def stepn(g, carry):
        for b in range(NBUF):
            s = g * NBUF + b
            issue_gathers(s + NBUF - 1, (b + NBUF - 1) % NBUF)
            wait_gathers(b)
            @pl.when(s >= NBUF)
            def _():
                pltpu.make_async_copy(
                    out_v.at[b], m_hbm.at[pl.ds(base, CB)], sem_out[b]).wait()
            for n in range(CB):
                for cb in range(C // L):
                    sl = pl.ds(cb * L, L)
                    acc = rows0_v[b, n * K, sl] - rows1_v[b, n * K, sl]
                    for k in range(1, K):
                        acc = jnp.maximum(
                            acc, rows0_v[b, n * K + k, sl] - rows1_v[b, n * K + k, sl])
                    out_v[b, n, sl] = acc
            pltpu.async_copy(out_v.at[b], m_hbm.at[pl.ds(base + s * CB, CB)], sem_out[b])
        return carry

    lax.fori_loop(0, STEPS // NBUF, stepn, 0)

    for b in range(NBUF - 1):
        wait_gathers(b)
    for b in range(NBUF):
        pltpu.make_async_copy(
            out_v.at[b], m_hbm.at[pl.ds(base, CB)], sem_out[b]).wait()


@functools.partial(jax.jit, static_argnames=())
def _sc_max_rel(xt, idx0_flat, idx1_flat):
    mesh = plsc.VectorSubcoreMesh(core_axis_name="c", subcore_axis_name="s")
    f = functools.partial(
        pl.kernel,
        mesh=mesh,
        compiler_params=pltpu.CompilerParams(needs_layout_passes=False),
        out_type=jax.ShapeDtypeStruct((NPAD, C), jnp.float32),
        scratch_types=[
            pltpu.VMEM(((NPW + (NBUF - 1) * CB) * K,), jnp.int32),
            pltpu.VMEM(((NPW + (NBUF - 1) * CB) * K,), jnp.int32),
            pltpu.VMEM((NBUF, G, C), jnp.float32),
            pltpu.VMEM((NBUF, G, C), jnp.float32),
            pltpu.VMEM((NBUF, CB, C), jnp.float32),
            pltpu.SemaphoreType.DMA,
            pltpu.SemaphoreType.DMA,
            pltpu.SemaphoreType.DMA,
            pltpu.SemaphoreType.DMA,
            pltpu.SemaphoreType.DMA,
            pltpu.SemaphoreType.DMA,
            pltpu.SemaphoreType.DMA,
            pltpu.SemaphoreType.DMA,
        ],
    )(_sc_body)
    return f(xt, idx0_flat, idx1_flat)


def _tc_body(xt_ref, m_ref, w_ref, b_ref, out_ref):
    w1 = w_ref[:, :C]
    w2 = w_ref[:, C:]
    acc = lax.dot_general(w1, xt_ref[...], (((1,), (1,)), ((), ())),
                          preferred_element_type=jnp.float32,
                          precision=lax.Precision.HIGHEST)
    acc += lax.dot_general(w2, m_ref[...], (((1,), (1,)), ((), ())),
                           preferred_element_type=jnp.float32,
                           precision=lax.Precision.HIGHEST)
    out_ref[...] = jnp.maximum(acc + b_ref[...], 0.0)


NB = 512


def _tc_fuse(xt, m, w, b2):
    grid = (NPAD // NB,)
    return pl.pallas_call(
        _tc_body,
        grid=grid,
        in_specs=[
            pl.BlockSpec((NB, C), lambda j: (j, 0)),
            pl.BlockSpec((NB, C), lambda j: (j, 0)),
            pl.BlockSpec((C_OUT, 2 * C), lambda j: (0, 0)),
            pl.BlockSpec((C_OUT, 1), lambda j: (0, 0)),
        ],
        out_specs=pl.BlockSpec((C_OUT, NB), lambda j: (0, j)),
        out_shape=jax.ShapeDtypeStruct((C_OUT, NPAD), jnp.float32),
    )(xt, m, w, b2)


def kernel(x, edge_index, att_w, att_b, conv_w, conv_b):
    xc = x[0, :, :, 0]
    pad = NPAD - N
    xt = jnp.pad(jnp.transpose(xc), ((0, pad), (0, 0)))
    e0 = edge_index[0, 0].astype(jnp.int32)
    e1 = edge_index[1, 0].astype(jnp.int32)
    idx0 = jnp.pad(e0, ((0, pad + (NBUF - 1) * CB), (0, 0))).reshape(-1)
    idx1 = jnp.pad(e1, ((0, pad + (NBUF - 1) * CB), (0, 0))).reshape(-1)
    m = _sc_max_rel(xt, idx0, idx1)
    wi = conv_w[:, :, 0, 0]
    w = jnp.concatenate([wi[:, 0::2], wi[:, 1::2]], axis=1)
    b2 = conv_b[:, None]
    out = _tc_fuse(xt, m, w, b2)
    return out[None, :, :N, None]


# speedup vs baseline: 1.4383x; 1.4383x over previous
import functools

import jax
import jax.numpy as jnp
from jax import lax
from jax.experimental import pallas as pl
from jax.experimental.pallas import tpu as pltpu
from jax.experimental.pallas import tpu_sc as plsc

N = 50000
C = 128
K = 12
C_OUT = 128

NC = 2
NS = 16
NW = NC * NS
L = 16

CB = 2
NPW = 1568
STEPS = NPW // CB
NPAD = NW * NPW
G = CB * K
H = G // 2


NBUF = 8


def _sc_body(xt_hbm, idx0_hbm, idx1_hbm, m_hbm,
             idx0_v, idx1_v, rows0_v, rows1_v, out_v,
             sg0, sg1, sg2, sg3, sg4, sg5, sg6, sg7,
             so0, so1, so2, so3, so4, so5, so6, so7):
    wid = lax.axis_index("s") * NC + lax.axis_index("c")
    base = wid * NPW
    sem_g = (sg0, sg1, sg2, sg3, sg4, sg5, sg6, sg7)
    sem_out = (so0, so1, so2, so3, so4, so5, so6, so7)

    pltpu.sync_copy(idx0_hbm.at[pl.ds(base * K, (NPW + (NBUF - 1) * CB) * K)], idx0_v)
    pltpu.sync_copy(idx1_hbm.at[pl.ds(base * K, (NPW + (NBUF - 1) * CB) * K)], idx1_v)

    def issue_gathers(s, b):
        off = s * G
        pltpu.async_copy(xt_hbm.at[idx0_v.at[pl.ds(off, G)]], rows0_v.at[b], sem_g[b])
        pltpu.async_copy(xt_hbm.at[idx1_v.at[pl.ds(off, G)]], rows1_v.at[b], sem_g[b])

    def wait_gathers(b):
        pltpu.make_async_copy(xt_hbm.at[idx0_v.at[pl.ds(0, G)]], rows0_v.at[b], sem_g[b]).wait()
        pltpu.make_async_copy(xt_hbm.at[idx1_v.at[pl.ds(0, G)]], rows1_v.at[b], sem_g[b]).wait()

    for b in range(NBUF - 1):
        issue_gathers(b, b)

    def stepn(g, carry):
        for b in range(NBUF):
            s = g * NBUF + b
            issue_gathers(s + NBUF - 1, (b + NBUF - 1) % NBUF)
            wait_gathers(b)
            @pl.when(s >= NBUF)
            def _():
                pltpu.make_async_copy(
                    out_v.at[b], m_hbm.at[pl.ds(base, CB)], sem_out[b]).wait()
            for n in range(CB):
                sls = [pl.ds(cb * L, L) for cb in range(C // L)]

                def kbody(k, accs, n=n, sls=sls):
                    r = n * K + k
                    return tuple(
                        jnp.maximum(a, rows0_v[b, r, sl] - rows1_v[b, r, sl])
                        for a, sl in zip(accs, sls))

                accs = tuple(
                    rows0_v[b, n * K, sl] - rows1_v[b, n * K, sl] for sl in sls)
                accs = lax.fori_loop(1, K, kbody, accs)
                for acc, sl in zip(accs, sls):
                    out_v[b, n, sl] = acc
            pltpu.async_copy(out_v.at[b], m_hbm.at[pl.ds(base + s * CB, CB)], sem_out[b])
        return carry

    lax.fori_loop(0, STEPS // NBUF, stepn, 0)

    for b in range(NBUF - 1):
        wait_gathers(b)
    for b in range(NBUF):
        pltpu.make_async_copy(
            out_v.at[b], m_hbm.at[pl.ds(base, CB)], sem_out[b]).wait()


@functools.partial(jax.jit, static_argnames=())
def _sc_max_rel(xt, idx0_flat, idx1_flat):
    mesh = plsc.VectorSubcoreMesh(core_axis_name="c", subcore_axis_name="s")
    f = functools.partial(
        pl.kernel,
        mesh=mesh,
        compiler_params=pltpu.CompilerParams(needs_layout_passes=False),
        out_type=jax.ShapeDtypeStruct((NPAD, C), jnp.float32),
        scratch_types=[
            pltpu.VMEM(((NPW + (NBUF - 1) * CB) * K,), jnp.int32),
            pltpu.VMEM(((NPW + (NBUF - 1) * CB) * K,), jnp.int32),
            pltpu.VMEM((NBUF, G, C), jnp.float32),
            pltpu.VMEM((NBUF, G, C), jnp.float32),
            pltpu.VMEM((NBUF, CB, C), jnp.float32),
        ] + [pltpu.SemaphoreType.DMA] * (2 * NBUF),
    )(_sc_body)
    return f(xt, idx0_flat, idx1_flat)


def _tc_body(xt_ref, m_ref, w_ref, b_ref, out_ref):
    w1 = w_ref[:, :C]
    w2 = w_ref[:, C:]
    acc = lax.dot_general(w1, xt_ref[...], (((1,), (1,)), ((), ())),
                          preferred_element_type=jnp.float32,
                          precision=lax.Precision.HIGHEST)
    acc += lax.dot_general(w2, m_ref[...], (((1,), (1,)), ((), ())),
                           preferred_element_type=jnp.float32,
                           precision=lax.Precision.HIGHEST)
    out_ref[...] = jnp.maximum(acc + b_ref[...], 0.0)


NB = 512


def _tc_fuse(xt, m, w, b2):
    grid = (NPAD // NB,)
    return pl.pallas_call(
        _tc_body,
        grid=grid,
        in_specs=[
            pl.BlockSpec((NB, C), lambda j: (j, 0)),
            pl.BlockSpec((NB, C), lambda j: (j, 0)),
            pl.BlockSpec((C_OUT, 2 * C), lambda j: (0, 0)),
            pl.BlockSpec((C_OUT, 1), lambda j: (0, 0)),
        ],
        out_specs=pl.BlockSpec((C_OUT, NB), lambda j: (0, j)),
        out_shape=jax.ShapeDtypeStruct((C_OUT, NPAD), jnp.float32),
    )(xt, m, w, b2)


def kernel(x, edge_index, att_w, att_b, conv_w, conv_b):
    xc = x[0, :, :, 0]
    pad = NPAD - N
    xt = jnp.pad(jnp.transpose(xc), ((0, pad), (0, 0)))
    e0 = edge_index[0, 0].astype(jnp.int32)
    e1 = edge_index[1, 0].astype(jnp.int32)
    idx0 = jnp.pad(e0, ((0, pad + (NBUF - 1) * CB), (0, 0))).reshape(-1)
    idx1 = jnp.pad(e1, ((0, pad + (NBUF - 1) * CB), (0, 0))).reshape(-1)
    m = _sc_max_rel(xt, idx0, idx1)
    wi = conv_w[:, :, 0, 0]
    w = jnp.concatenate([wi[:, 0::2], wi[:, 1::2]], axis=1)
    b2 = conv_b[:, None]
    out = _tc_fuse(xt, m, w, b2)
    return out[None, :, :N, None]
